# baseline (speedup 1.0000x reference)
import jax
import jax.numpy as jnp
from jax.experimental import pallas as pl
from jax.experimental.pallas import tpu as pltpu

TOTAL = 32768
D = 256
N_OUT = TOTAL - 2
RS_OUT = TOTAL - 1
FLAT = N_OUT * D

ROWS = [1024, 2048, 4096, 8192, 8192, 8192, 1022]
assert sum(ROWS) == N_OUT
CHUNKS = []
_off = 0
for _r in ROWS:
    CHUNKS.append((_off * D, _r * D))
    _off += _r


def _copy_body(x_ref, rs_ref, data_ref, rs_out_ref, *scratch):
    n = len(CHUNKS)
    bufs = scratch[:n]
    lsems = scratch[n:2 * n]
    ssems = scratch[2 * n:3 * n]
    loads, stores = [], []
    for k, (off, sz) in enumerate(CHUNKS):
        cp = pltpu.make_async_copy(x_ref.at[pl.ds(off, sz)], bufs[k], lsems[k])
        cp.start()
        loads.append(cp)
    rs_out_ref[...] = rs_ref[pl.ds(0, RS_OUT)]
    for k, (off, sz) in enumerate(CHUNKS):
        loads[k].wait()
        cp = pltpu.make_async_copy(bufs[k], data_ref.at[pl.ds(off, sz)], ssems[k])
        cp.start()
        stores.append(cp)
    for cp in stores:
        cp.wait()


def kernel(x_data, x_row_splits):
    n = len(CHUNKS)
    data_flat, rs = pl.pallas_call(
        _copy_body,
        in_specs=[
            pl.BlockSpec(memory_space=pltpu.MemorySpace.HBM),
            pl.BlockSpec((TOTAL,), lambda: (0,)),
        ],
        out_specs=[
            pl.BlockSpec(memory_space=pltpu.MemorySpace.HBM),
            pl.BlockSpec((RS_OUT,), lambda: (0,)),
        ],
        out_shape=[
            jax.ShapeDtypeStruct((FLAT,), jnp.float32),
            jax.ShapeDtypeStruct((RS_OUT,), jnp.int32),
        ],
        scratch_shapes=(
            [pltpu.VMEM((sz,), jnp.float32) for _, sz in CHUNKS]
            + [pltpu.SemaphoreType.DMA] * n
            + [pltpu.SemaphoreType.DMA] * n
        ),
    )(x_data.reshape(-1), x_row_splits)
    return (data_flat.reshape(N_OUT, D), rs)


# speedup vs baseline: 4.2855x; 4.2855x over previous
import jax
import jax.numpy as jnp
from jax.experimental import pallas as pl
from jax.experimental.pallas import tpu as pltpu

TOTAL = 32768
D = 256
N_OUT = TOTAL - 2
RS_OUT = TOTAL - 1
ALIGNED = 32760
TAIL = N_OUT - ALIGNED

ROWS = [1024, 2048, 4096, 8184, 8184, 8184, 1040]
assert sum(ROWS) == ALIGNED and all(r % 8 == 0 for r in ROWS)
OFFS = []
_o = 0
for _r in ROWS:
    OFFS.append(_o)
    _o += _r


def _copy_body(x_ref, xtail_ref, rs_ref, data_ref, tail_ref, rs_out_ref, *scratch):
    n = len(ROWS)
    bufs = scratch[:n]
    lsems = scratch[n:2 * n]
    ssems = scratch[2 * n:3 * n]
    loads = []
    for k in range(n):
        cp = pltpu.make_async_copy(
            x_ref.at[pl.ds(OFFS[k], ROWS[k])], bufs[k], lsems[k])
        cp.start()
        loads.append(cp)
    tail_ref[...] = xtail_ref[...]
    rs_out_ref[...] = rs_ref[pl.ds(0, RS_OUT)]
    stores = []
    for k in range(n):
        loads[k].wait()
        cp = pltpu.make_async_copy(
            bufs[k], data_ref.at[pl.ds(OFFS[k], ROWS[k])], ssems[k])
        cp.start()
        stores.append(cp)
    for cp in stores:
        cp.wait()


def kernel(x_data, x_row_splits):
    n = len(ROWS)
    data, tail8, rs = pl.pallas_call(
        _copy_body,
        grid=(1,),
        in_specs=[
            pl.BlockSpec(memory_space=pltpu.MemorySpace.HBM),
            pl.BlockSpec((8, D), lambda i: (TOTAL // 8 - 1, 0)),
            pl.BlockSpec((TOTAL,), lambda i: (0,)),
        ],
        out_specs=[
            pl.BlockSpec(memory_space=pltpu.MemorySpace.HBM),
            pl.BlockSpec((8, D), lambda i: (0, 0)),
            pl.BlockSpec((RS_OUT,), lambda i: (0,)),
        ],
        out_shape=[
            jax.ShapeDtypeStruct((N_OUT, D), jnp.float32),
            jax.ShapeDtypeStruct((8, D), jnp.float32),
            jax.ShapeDtypeStruct((RS_OUT,), jnp.int32),
        ],
        scratch_shapes=(
            [pltpu.VMEM((r, D), jnp.float32) for r in ROWS]
            + [pltpu.SemaphoreType.DMA] * n
            + [pltpu.SemaphoreType.DMA] * n
        ),
    )(x_data, x_data, x_row_splits)
    data = jax.lax.dynamic_update_slice(
        data, jax.lax.slice(tail8, (0, 0), (TAIL, D)), (ALIGNED, 0))
    return (data, rs)


# speedup vs baseline: 5.0795x; 1.1853x over previous
import jax
import jax.numpy as jnp
from jax.experimental import pallas as pl
from jax.experimental.pallas import tpu as pltpu

TOTAL = 32768
D = 256
N_OUT = TOTAL - 2
RS_OUT = TOTAL - 1
BLK = 14936


def _copy_body(x_ref, rs_ref, data_ref, rs_out_ref):
    data_ref[...] = x_ref[...]
    i = pl.program_id(0)

    @pl.when(i == 0)
    def _():
        rs_out_ref[...] = rs_ref[pl.ds(0, RS_OUT)]


def kernel(x_data, x_row_splits):
    grid = (pl.cdiv(N_OUT, BLK),)
    data, rs = pl.pallas_call(
        _copy_body,
        grid=grid,
        in_specs=[
            pl.BlockSpec((BLK, D), lambda i: (i, 0)),
            pl.BlockSpec((TOTAL,), lambda i: (0,)),
        ],
        out_specs=[
            pl.BlockSpec((BLK, D), lambda i: (i, 0)),
            pl.BlockSpec((RS_OUT,), lambda i: (0,)),
        ],
        out_shape=[
            jax.ShapeDtypeStruct((N_OUT, D), jnp.float32),
            jax.ShapeDtypeStruct((RS_OUT,), jnp.int32),
        ],
    )(x_data, x_row_splits)
    return (data, rs)


# speedup vs baseline: 5.0840x; 1.0009x over previous
import jax
import jax.numpy as jnp
from jax.experimental import pallas as pl

TOTAL = 32768
D = 256
N_OUT = TOTAL - 2
RS_OUT = TOTAL - 1
BLK = 14936


def _copy_body(x_ref, rs_ref, data_ref, rs_out_ref):
    data_ref[...] = x_ref[...]
    i = pl.program_id(0)

    @pl.when(i == 0)
    def _():
        rs_out_ref[...] = rs_ref[pl.ds(0, RS_OUT)]


def kernel(x_data, x_row_splits):
    grid = (pl.cdiv(N_OUT, BLK),)
    data, rs = pl.pallas_call(
        _copy_body,
        grid=grid,
        in_specs=[
            pl.BlockSpec((BLK, D), lambda i: (i, 0)),
            pl.BlockSpec((TOTAL,), lambda i: (0,)),
        ],
        out_specs=[
            pl.BlockSpec((BLK, D), lambda i: (i, 0)),
            pl.BlockSpec((RS_OUT,), lambda i: (0,)),
        ],
        out_shape=[
            jax.ShapeDtypeStruct((N_OUT, D), jnp.float32),
            jax.ShapeDtypeStruct((RS_OUT,), jnp.int32),
        ],
    )(x_data, x_row_splits)
    return (data, rs)
